# baseline (speedup 1.0000x reference)
import jax
import jax.numpy as jnp
from jax import lax
from jax.experimental import pallas as pl
from jax.experimental.pallas import tpu as pltpu, tpu_sc as plsc

USER = 25000
ITEM = 25000
N = USER + ITEM
E = 800000
LATDIM = 64
HALF = 32

NTILES = 16
CHUNK = 128
JCHUNKS = 8
GCHUNKS = 50
EDGES_PER_TILE = CHUNK * JCHUNKS * GCHUNKS
E_PAD = EDGES_PER_TILE * NTILES
N_PAD = 51200
ROWS_PER_TILE = N_PAD // NTILES
RCHUNKS = ROWS_PER_TILE // CHUNK
BOT_PAD = 25088
BOT_PER_TILE = BOT_PAD // NTILES


def _mega_body(fimg, ftxt, base, botimg, bottxt, zeros_hbm, ident,
               asrc, adst, isrc, idst, tsrc, tdst,
               v_adj, v_adj_w0, v_adj_w1, v_iadj, v_tadj,
               out, m, x3i, x3t, t1, g1,
               acc, sidx, didx, vbuf, rows, vident,
               gsem0, gsem1, gsem2, gsem3, isem_s, isem_d, isem_v, tsem):
    c = lax.axis_index("c")
    s = lax.axis_index("s")
    gsems = (gsem0, gsem1, gsem2, gsem3)
    row0 = s * (JCHUNKS * GCHUNKS)
    arow = s * ROWS_PER_TILE

    def zero_acc():
        pltpu.sync_copy(zeros_hbm.at[pl.ds(arow, ROWS_PER_TILE)],
                        acc.at[pl.ds(arow, ROWS_PER_TILE)])

    def wb_acc(dst_hbm):
        pltpu.sync_copy(acc.at[pl.ds(arow, ROWS_PER_TILE)],
                        dst_hbm.at[c].at[pl.ds(arow, ROWS_PER_TILE)])

    def wb_top(dst_hbm):
        @pl.when(s < USER // ROWS_PER_TILE)
        def _():
            pltpu.sync_copy(acc.at[pl.ds(arow, ROWS_PER_TILE)],
                            dst_hbm.at[c].at[pl.ds(arow, ROWS_PER_TILE)])

        @pl.when(s == USER // ROWS_PER_TILE)
        def _():
            rem = USER % ROWS_PER_TILE
            pltpu.sync_copy(acc.at[pl.ds(arow, rem)],
                            dst_hbm.at[c].at[pl.ds(arow, rem)])

    def spmm(src_hbm, dst_hbm, val_hbm, x_hbm):
        def stage_idx(g, p):
            r0 = row0 + g * JCHUNKS
            pltpu.async_copy(src_hbm.at[pl.ds(r0, JCHUNKS)], sidx.at[p], isem_s)
            pltpu.async_copy(dst_hbm.at[pl.ds(r0, JCHUNKS)], didx.at[p], isem_d)
            pltpu.async_copy(val_hbm.at[pl.ds(r0, JCHUNKS)], vbuf.at[p], isem_v)

        def wait_idx(p):
            pltpu.make_async_copy(src_hbm.at[pl.ds(row0, JCHUNKS)], sidx.at[p], isem_s).wait()
            pltpu.make_async_copy(dst_hbm.at[pl.ds(row0, JCHUNKS)], didx.at[p], isem_d).wait()
            pltpu.make_async_copy(val_hbm.at[pl.ds(row0, JCHUNKS)], vbuf.at[p], isem_v).wait()

        def fire_gather(p, j, q):
            pltpu.async_copy(x_hbm.at[c].at[sidx.at[p].at[j]], rows.at[q], gsems[q])

        def wait_gather(p, j, q):
            pltpu.make_async_copy(x_hbm.at[c].at[sidx.at[p].at[j]], rows.at[q],
                                  gsems[q]).wait()

        def scale_rows(p, j, q):
            def scale16(b, _):
                val16 = vbuf[p, j, pl.ds(b * 16, 16)]
                for t in range(16):
                    v = jnp.full((16,), val16[t], dtype=jnp.float32)
                    r = b * 16 + t
                    rows[q, r, 0:16] = rows[q, r, 0:16] * v
                    rows[q, r, 16:32] = rows[q, r, 16:32] * v
                return ()
            lax.fori_loop(0, CHUNK // 16, scale16, ())

        stage_idx(0, 0)

        def super_chunk(g, _):
            p = lax.rem(g, 2)
            wait_idx(p)

            @pl.when(g < GCHUNKS - 1)
            def _():
                stage_idx(g + 1, 1 - p)

            for q in range(4):
                fire_gather(p, q, q)

            def group(k, _):
                for q in range(4):
                    j = k * 4 + q
                    wait_gather(p, j, q)
                    scale_rows(p, j, q)
                    pltpu.sync_copy(rows.at[q], acc.at[didx.at[p].at[j]], add=True)

                    @pl.when(k < JCHUNKS // 4 - 1)
                    def _():
                        fire_gather(p, j + 4, q)
                return ()

            lax.fori_loop(0, JCHUNKS // 4, group, ())
            return ()

        lax.fori_loop(0, GCHUNKS, super_chunk, ())

    def ident_add(src_hbm):
        def body(k, _):
            pltpu.sync_copy(src_hbm.at[c].at[pl.ds(arow + k * CHUNK, CHUNK)], rows.at[0])
            pltpu.sync_copy(rows.at[0], acc.at[vident.at[k]], add=True)
            return ()
        lax.fori_loop(0, RCHUNKS, body, ())

    barrier = plsc.subcore_barrier

    pltpu.sync_copy(ident.at[pl.ds(s * RCHUNKS, RCHUNKS)], vident)

    bot0 = s * BOT_PER_TILE
    pltpu.sync_copy(botimg.at[c].at[pl.ds(bot0, BOT_PER_TILE)],
                    x3i.at[c].at[pl.ds(USER + bot0, BOT_PER_TILE)])
    pltpu.sync_copy(bottxt.at[c].at[pl.ds(bot0, BOT_PER_TILE)],
                    x3t.at[c].at[pl.ds(USER + bot0, BOT_PER_TILE)])

    zero_acc()
    barrier()
    spmm(asrc, adst, v_adj_w1, ftxt)
    barrier()
    wb_acc(t1)
    wb_top(x3t)
    zero_acc()
    barrier()
    spmm(asrc, adst, v_adj_w0, fimg)
    barrier()
    wb_top(x3i)
    barrier()
    spmm(asrc, adst, v_adj, x3i)
    spmm(isrc, idst, v_iadj, base)
    spmm(asrc, adst, v_adj, x3t)
    spmm(tsrc, tdst, v_tadj, base)
    ident_add(t1)
    barrier()
    wb_acc(m)
    zero_acc()
    barrier()
    spmm(asrc, adst, v_adj, m)
    barrier()
    wb_acc(g1)
    barrier()
    spmm(asrc, adst, v_adj, g1)
    ident_add(m)
    barrier()
    wb_acc(out)


_abst = jax.ShapeDtypeStruct((2, N_PAD, HALF), jnp.float32)
_mega_call = pl.kernel(
    _mega_body,
    out_type=(_abst,) * 6,
    mesh=plsc.VectorSubcoreMesh(core_axis_name="c", subcore_axis_name="s"),
    scratch_types=[
        pltpu.VMEM_SHARED((N_PAD, HALF), jnp.float32),
        pltpu.VMEM((2, JCHUNKS, CHUNK), jnp.int32),
        pltpu.VMEM((2, JCHUNKS, CHUNK), jnp.int32),
        pltpu.VMEM((2, JCHUNKS, CHUNK), jnp.float32),
        pltpu.VMEM((4, CHUNK, HALF), jnp.float32),
        pltpu.VMEM((RCHUNKS, CHUNK), jnp.int32),
    ] + [pltpu.SemaphoreType.DMA] * 8,
    compiler_params=pltpu.CompilerParams(use_tc_tiling_on_sc=False),
)


def _prep_idx(idx):
    pad = E_PAD - E
    fill = jnp.arange(pad, dtype=jnp.int32)
    src = jnp.concatenate([idx[1], fill % N]).reshape(-1, CHUNK)
    dst = jnp.concatenate([idx[0], N + fill % (N_PAD - N)]).reshape(-1, CHUNK)
    return src, dst


def _prep_val(vals):
    return jnp.pad(vals, (0, E_PAD - E)).reshape(-1, CHUNK)


def _to_half(x):
    return x.reshape(x.shape[0], 2, HALF).transpose(1, 0, 2)


def _from_half(x2):
    return x2.transpose(1, 0, 2).reshape(x2.shape[1], LATDIM)


def _l2norm(x):
    n = jnp.linalg.norm(x, axis=1, keepdims=True)
    return x / jnp.maximum(n, 1e-12)


def _final_kernel(pre_ref, m_ref, o_ref):
    mm = m_ref[...]
    sq = jnp.sum(mm * mm, axis=1, keepdims=True)
    o_ref[...] = pre_ref[...] + 0.2 * (mm / jnp.maximum(jnp.sqrt(sq), 1e-12))


def kernel(uEmbeds, iEmbeds, image_embedding, text_embedding, W_img, b_img, W_txt, b_txt,
           modal_weight, adj_vals, image_adj_vals, text_adj_vals,
           adj_idx, image_adj_idx, text_adj_idx):
    weight = jax.nn.softmax(modal_weight, axis=0)
    w0, w1 = weight[0], weight[1]

    asrc, adst = _prep_idx(adj_idx)
    isrc, idst = _prep_idx(image_adj_idx)
    tsrc, tdst = _prep_idx(text_adj_idx)
    v_adj = _prep_val(adj_vals)
    v_adj_w0 = _prep_val(adj_vals * w0)
    v_adj_w1 = _prep_val(adj_vals * w1)
    v_iadj = _prep_val(image_adj_vals * (0.2 * w0))
    v_tadj = _prep_val(text_adj_vals * (0.2 * w1))

    image_feats = image_embedding @ W_img + b_img
    text_feats = text_embedding @ W_txt + b_txt

    u2 = _to_half(uEmbeds)
    i2 = _to_half(iEmbeds)
    base2 = jnp.concatenate([u2, i2], axis=1)
    fimg = jnp.concatenate([u2, _to_half(_l2norm(image_feats))], axis=1)
    ftxt = jnp.concatenate([u2, _to_half(_l2norm(text_feats))], axis=1)
    botimg = jnp.pad(w0 * i2, ((0, 0), (0, BOT_PAD - ITEM), (0, 0)))
    bottxt = jnp.pad(w1 * i2, ((0, 0), (0, BOT_PAD - ITEM), (0, 0)))
    zeros = jnp.zeros((N_PAD, HALF), jnp.float32)
    ident = jnp.arange(N_PAD, dtype=jnp.int32).reshape(-1, CHUNK)

    out2, m2, _, _, _, _ = _mega_call(
        fimg, ftxt, base2, botimg, bottxt, zeros, ident,
        asrc, adst, isrc, idst, tsrc, tdst,
        v_adj, v_adj_w0, v_adj_w1, v_iadj, v_tadj)

    pre = _from_half(out2[:, :N])
    mfull = _from_half(m2[:, :N])

    BM = 1000
    embeds = pl.pallas_call(
        _final_kernel,
        grid=(N // BM,),
        in_specs=[pl.BlockSpec((BM, LATDIM), lambda i: (i, 0))] * 2,
        out_specs=pl.BlockSpec((BM, LATDIM), lambda i: (i, 0)),
        out_shape=jax.ShapeDtypeStruct((N, LATDIM), jnp.float32),
    )(pre, mfull)
    return embeds[:USER], embeds[USER:]


# speedup vs baseline: 1.1491x; 1.1491x over previous
import jax
import jax.numpy as jnp
from jax import lax
from jax.experimental import pallas as pl
from jax.experimental.pallas import tpu as pltpu, tpu_sc as plsc

USER = 25000
ITEM = 25000
N = USER + ITEM
E = 800000
LATDIM = 64
HALF = 32

NTILES = 16
CHUNK = 128
JCHUNKS = 8
GCHUNKS = 50
EDGES_PER_TILE = CHUNK * JCHUNKS * GCHUNKS
E_PAD = EDGES_PER_TILE * NTILES
N_PAD = 51200
ROWS_PER_TILE = N_PAD // NTILES
RCHUNKS = ROWS_PER_TILE // CHUNK
BOT_PAD = 25088
BOT_PER_TILE = BOT_PAD // NTILES


def _mega_body(fimg, ftxt, base, botimg, bottxt, zeros_hbm, ident,
               asrc, adst, isrc, idst, tsrc, tdst,
               v_adj, v_adj_w0, v_adj_w1, v_iadj, v_tadj,
               out, m, x3i, x3t, t1, g1,
               acc, sidx, didx, vbuf, rows, vident,
               gsem0, gsem1, gsem2, gsem3, isem_s, isem_d, isem_v, tsem):
    c = lax.axis_index("c")
    s = lax.axis_index("s")
    gsems = (gsem0, gsem1, gsem2, gsem3)
    row0 = s * (JCHUNKS * GCHUNKS)
    arow = s * ROWS_PER_TILE

    def zero_acc():
        pltpu.sync_copy(zeros_hbm.at[pl.ds(arow, ROWS_PER_TILE)],
                        acc.at[pl.ds(arow, ROWS_PER_TILE)])

    def wb_acc(dst_hbm):
        pltpu.sync_copy(acc.at[pl.ds(arow, ROWS_PER_TILE)],
                        dst_hbm.at[c].at[pl.ds(arow, ROWS_PER_TILE)])

    def wb_top(dst_hbm):
        @pl.when(s < USER // ROWS_PER_TILE)
        def _():
            pltpu.sync_copy(acc.at[pl.ds(arow, ROWS_PER_TILE)],
                            dst_hbm.at[c].at[pl.ds(arow, ROWS_PER_TILE)])

        @pl.when(s == USER // ROWS_PER_TILE)
        def _():
            rem = USER % ROWS_PER_TILE
            pltpu.sync_copy(acc.at[pl.ds(arow, rem)],
                            dst_hbm.at[c].at[pl.ds(arow, rem)])

    def spmm(src_hbm, dst_hbm, val_hbm, x_hbm):
        TOT = GCHUNKS * JCHUNKS

        def stage_idx(k, p):
            r0 = row0 + k * JCHUNKS
            pltpu.async_copy(src_hbm.at[pl.ds(r0, JCHUNKS)], sidx.at[p], isem_s)
            pltpu.async_copy(dst_hbm.at[pl.ds(r0, JCHUNKS)], didx.at[p], isem_d)
            pltpu.async_copy(val_hbm.at[pl.ds(r0, JCHUNKS)], vbuf.at[p], isem_v)

        def wait_idx(p):
            pltpu.make_async_copy(src_hbm.at[pl.ds(row0, JCHUNKS)], sidx.at[p], isem_s).wait()
            pltpu.make_async_copy(dst_hbm.at[pl.ds(row0, JCHUNKS)], didx.at[p], isem_d).wait()
            pltpu.make_async_copy(val_hbm.at[pl.ds(row0, JCHUNKS)], vbuf.at[p], isem_v).wait()

        def fire_gather(g):
            p = lax.rem(g // JCHUNKS, 2)
            jj = lax.rem(g, JCHUNKS)
            q = lax.rem(g, 4)
            pltpu.async_copy(x_hbm.at[c].at[sidx.at[p].at[jj]], rows.at[q],
                             gsems_arr(q))

        def gsems_arr(q):
            return gsems[0] if isinstance(q, int) else None

        def chunk_step(g, _):
            k = g // JCHUNKS
            jj = lax.rem(g, JCHUNKS)
            p = lax.rem(k, 2)

            @pl.when(jnp.logical_and(jj == 0, g < TOT - JCHUNKS))
            def _():
                stage_idx(k + 1, 1 - p)

            @pl.when(jnp.logical_and(jj == 3, g < TOT - JCHUNKS + 3))
            def _():
                wait_idx(1 - p)

            for q in range(4):
                @pl.when(lax.rem(g, 4) == q)
                def _():
                    pltpu.make_async_copy(x_hbm.at[c].at[sidx.at[p].at[jj]],
                                          rows.at[q], gsems[q]).wait()

                    def scale16(b, _):
                        val16 = vbuf[p, jj, pl.ds(b * 16, 16)]
                        for t in range(16):
                            v = jnp.full((16,), val16[t], dtype=jnp.float32)
                            r = b * 16 + t
                            rows[q, r, 0:16] = rows[q, r, 0:16] * v
                            rows[q, r, 16:32] = rows[q, r, 16:32] * v
                        return ()

                    lax.fori_loop(0, CHUNK // 16, scale16, ())
                    pltpu.sync_copy(rows.at[q], acc.at[didx.at[p].at[jj]], add=True)

                    @pl.when(g < TOT - 4)
                    def _():
                        g4 = g + 4
                        p4 = lax.rem(g4 // JCHUNKS, 2)
                        jj4 = lax.rem(g4, JCHUNKS)
                        pltpu.async_copy(x_hbm.at[c].at[sidx.at[p4].at[jj4]],
                                         rows.at[q], gsems[q])
            return ()

        stage_idx(0, 0)
        wait_idx(0)
        for q in range(4):
            pltpu.async_copy(x_hbm.at[c].at[sidx.at[0].at[q]], rows.at[q], gsems[q])
        lax.fori_loop(0, TOT, chunk_step, ())

    def ident_add(src_hbm):
        def body(k, _):
            pltpu.sync_copy(src_hbm.at[c].at[pl.ds(arow + k * CHUNK, CHUNK)], rows.at[0])
            pltpu.sync_copy(rows.at[0], acc.at[vident.at[k]], add=True)
            return ()
        lax.fori_loop(0, RCHUNKS, body, ())

    barrier = plsc.subcore_barrier

    pltpu.sync_copy(ident.at[pl.ds(s * RCHUNKS, RCHUNKS)], vident)

    bot0 = s * BOT_PER_TILE
    pltpu.sync_copy(botimg.at[c].at[pl.ds(bot0, BOT_PER_TILE)],
                    x3i.at[c].at[pl.ds(USER + bot0, BOT_PER_TILE)])
    pltpu.sync_copy(bottxt.at[c].at[pl.ds(bot0, BOT_PER_TILE)],
                    x3t.at[c].at[pl.ds(USER + bot0, BOT_PER_TILE)])

    zero_acc()
    barrier()
    spmm(asrc, adst, v_adj_w1, ftxt)
    barrier()
    wb_acc(t1)
    wb_top(x3t)
    zero_acc()
    barrier()
    spmm(asrc, adst, v_adj_w0, fimg)
    barrier()
    wb_top(x3i)
    barrier()
    spmm(asrc, adst, v_adj, x3i)
    spmm(isrc, idst, v_iadj, base)
    spmm(asrc, adst, v_adj, x3t)
    spmm(tsrc, tdst, v_tadj, base)
    ident_add(t1)
    barrier()
    wb_acc(m)
    zero_acc()
    barrier()
    spmm(asrc, adst, v_adj, m)
    barrier()
    wb_acc(g1)
    barrier()
    spmm(asrc, adst, v_adj, g1)
    ident_add(m)
    barrier()
    wb_acc(out)


_abst = jax.ShapeDtypeStruct((2, N_PAD, HALF), jnp.float32)
_mega_call = pl.kernel(
    _mega_body,
    out_type=(_abst,) * 6,
    mesh=plsc.VectorSubcoreMesh(core_axis_name="c", subcore_axis_name="s"),
    scratch_types=[
        pltpu.VMEM_SHARED((N_PAD, HALF), jnp.float32),
        pltpu.VMEM((2, JCHUNKS, CHUNK), jnp.int32),
        pltpu.VMEM((2, JCHUNKS, CHUNK), jnp.int32),
        pltpu.VMEM((2, JCHUNKS, CHUNK), jnp.float32),
        pltpu.VMEM((4, CHUNK, HALF), jnp.float32),
        pltpu.VMEM((RCHUNKS, CHUNK), jnp.int32),
    ] + [pltpu.SemaphoreType.DMA] * 8,
    compiler_params=pltpu.CompilerParams(use_tc_tiling_on_sc=False),
)


def _prep_idx(idx):
    pad = E_PAD - E
    fill = jnp.arange(pad, dtype=jnp.int32)
    src = jnp.concatenate([idx[1], fill % N]).reshape(-1, CHUNK)
    dst = jnp.concatenate([idx[0], N + fill % (N_PAD - N)]).reshape(-1, CHUNK)
    return src, dst


def _prep_val(vals):
    return jnp.pad(vals, (0, E_PAD - E)).reshape(-1, CHUNK)


def _to_half(x):
    return x.reshape(x.shape[0], 2, HALF).transpose(1, 0, 2)


def _from_half(x2):
    return x2.transpose(1, 0, 2).reshape(x2.shape[1], LATDIM)


def _l2norm(x):
    n = jnp.linalg.norm(x, axis=1, keepdims=True)
    return x / jnp.maximum(n, 1e-12)


def _final_kernel(pre_ref, m_ref, o_ref):
    mm = m_ref[...]
    sq = jnp.sum(mm * mm, axis=1, keepdims=True)
    o_ref[...] = pre_ref[...] + 0.2 * (mm / jnp.maximum(jnp.sqrt(sq), 1e-12))


def kernel(uEmbeds, iEmbeds, image_embedding, text_embedding, W_img, b_img, W_txt, b_txt,
           modal_weight, adj_vals, image_adj_vals, text_adj_vals,
           adj_idx, image_adj_idx, text_adj_idx):
    weight = jax.nn.softmax(modal_weight, axis=0)
    w0, w1 = weight[0], weight[1]

    asrc, adst = _prep_idx(adj_idx)
    isrc, idst = _prep_idx(image_adj_idx)
    tsrc, tdst = _prep_idx(text_adj_idx)
    v_adj = _prep_val(adj_vals)
    v_adj_w0 = _prep_val(adj_vals * w0)
    v_adj_w1 = _prep_val(adj_vals * w1)
    v_iadj = _prep_val(image_adj_vals * (0.2 * w0))
    v_tadj = _prep_val(text_adj_vals * (0.2 * w1))

    image_feats = image_embedding @ W_img + b_img
    text_feats = text_embedding @ W_txt + b_txt

    u2 = _to_half(uEmbeds)
    i2 = _to_half(iEmbeds)
    base2 = jnp.concatenate([u2, i2], axis=1)
    fimg = jnp.concatenate([u2, _to_half(_l2norm(image_feats))], axis=1)
    ftxt = jnp.concatenate([u2, _to_half(_l2norm(text_feats))], axis=1)
    botimg = jnp.pad(w0 * i2, ((0, 0), (0, BOT_PAD - ITEM), (0, 0)))
    bottxt = jnp.pad(w1 * i2, ((0, 0), (0, BOT_PAD - ITEM), (0, 0)))
    zeros = jnp.zeros((N_PAD, HALF), jnp.float32)
    ident = jnp.arange(N_PAD, dtype=jnp.int32).reshape(-1, CHUNK)

    out2, m2, _, _, _, _ = _mega_call(
        fimg, ftxt, base2, botimg, bottxt, zeros, ident,
        asrc, adst, isrc, idst, tsrc, tdst,
        v_adj, v_adj_w0, v_adj_w1, v_iadj, v_tadj)

    pre = _from_half(out2[:, :N])
    mfull = _from_half(m2[:, :N])

    BM = 1000
    embeds = pl.pallas_call(
        _final_kernel,
        grid=(N // BM,),
        in_specs=[pl.BlockSpec((BM, LATDIM), lambda i: (i, 0))] * 2,
        out_specs=pl.BlockSpec((BM, LATDIM), lambda i: (i, 0)),
        out_shape=jax.ShapeDtypeStruct((N, LATDIM), jnp.float32),
    )(pre, mfull)
    return embeds[:USER], embeds[USER:]


# speedup vs baseline: 1.1496x; 1.0004x over previous
import jax
import jax.numpy as jnp
from jax import lax
from jax.experimental import pallas as pl
from jax.experimental.pallas import tpu as pltpu, tpu_sc as plsc

USER = 25000
ITEM = 25000
N = USER + ITEM
E = 800000
LATDIM = 64
HALF = 32

NTILES = 16
CHUNK = 128
JCHUNKS = 8
GCHUNKS = 50
EDGES_PER_TILE = CHUNK * JCHUNKS * GCHUNKS
E_PAD = EDGES_PER_TILE * NTILES
N_PAD = 51200
ROWS_PER_TILE = N_PAD // NTILES
RCHUNKS = ROWS_PER_TILE // CHUNK
BOT_PAD = 25088
BOT_PER_TILE = BOT_PAD // NTILES


def _mega_body(fimg, ftxt, base, botimg, bottxt, zeros_hbm, ident,
               asrc, adst, isrc, idst, tsrc, tdst,
               v_adj, v_adj_w0, v_adj_w1, v_iadj, v_tadj,
               out, m, x3i, x3t, t1, g1,
               acc, sidx, didx, vbuf, rows, vident,
               gsem0, gsem1, gsem2, gsem3, isem_s, isem_d, isem_v, tsem):
    c = lax.axis_index("c")
    s = lax.axis_index("s")
    gsems = (gsem0, gsem1, gsem2, gsem3)
    row0 = s * (JCHUNKS * GCHUNKS)
    arow = s * ROWS_PER_TILE

    def zero_acc():
        pltpu.sync_copy(zeros_hbm.at[pl.ds(arow, ROWS_PER_TILE)],
                        acc.at[pl.ds(arow, ROWS_PER_TILE)])

    def wb_acc(dst_hbm):
        pltpu.sync_copy(acc.at[pl.ds(arow, ROWS_PER_TILE)],
                        dst_hbm.at[c].at[pl.ds(arow, ROWS_PER_TILE)])

    def wb_top(dst_hbm):
        @pl.when(s < USER // ROWS_PER_TILE)
        def _():
            pltpu.sync_copy(acc.at[pl.ds(arow, ROWS_PER_TILE)],
                            dst_hbm.at[c].at[pl.ds(arow, ROWS_PER_TILE)])

        @pl.when(s == USER // ROWS_PER_TILE)
        def _():
            rem = USER % ROWS_PER_TILE
            pltpu.sync_copy(acc.at[pl.ds(arow, rem)],
                            dst_hbm.at[c].at[pl.ds(arow, rem)])

    def spmm(src_hbm, dst_hbm, val_hbm, x_hbm):
        TOT = GCHUNKS * JCHUNKS

        def stage_idx(k, p):
            r0 = row0 + k * JCHUNKS
            pltpu.async_copy(src_hbm.at[pl.ds(r0, JCHUNKS)], sidx.at[p], isem_s)
            pltpu.async_copy(dst_hbm.at[pl.ds(r0, JCHUNKS)], didx.at[p], isem_d)
            pltpu.async_copy(val_hbm.at[pl.ds(r0, JCHUNKS)], vbuf.at[p], isem_v)

        def wait_idx(p):
            pltpu.make_async_copy(src_hbm.at[pl.ds(row0, JCHUNKS)], sidx.at[p], isem_s).wait()
            pltpu.make_async_copy(dst_hbm.at[pl.ds(row0, JCHUNKS)], didx.at[p], isem_d).wait()
            pltpu.make_async_copy(val_hbm.at[pl.ds(row0, JCHUNKS)], vbuf.at[p], isem_v).wait()

        def chunk_step(g, _):
            k = g // JCHUNKS
            jj = lax.rem(g, JCHUNKS)
            p = lax.rem(k, 2)

            @pl.when(jnp.logical_and(jj == 0, g < TOT - JCHUNKS))
            def _():
                stage_idx(k + 1, 1 - p)

            @pl.when(jnp.logical_and(jj == 3, g < TOT - JCHUNKS + 3))
            def _():
                wait_idx(1 - p)

            for q in range(4):
                @pl.when(lax.rem(g, 4) == q)
                def _():
                    pltpu.make_async_copy(x_hbm.at[c].at[sidx.at[p].at[jj]],
                                          rows.at[q], gsems[q]).wait()

                    def scale16(b, _):
                        val16 = vbuf[p, jj, pl.ds(b * 16, 16)]
                        for t in range(16):
                            v = jnp.full((16,), val16[t], dtype=jnp.float32)
                            r = b * 16 + t
                            rows[q, r, 0:16] = rows[q, r, 0:16] * v
                            rows[q, r, 16:32] = rows[q, r, 16:32] * v
                        return ()

                    lax.fori_loop(0, CHUNK // 16, scale16, ())
                    pltpu.sync_copy(rows.at[q], acc.at[didx.at[p].at[jj]], add=True)

                    @pl.when(g < TOT - 4)
                    def _():
                        g4 = g + 4
                        p4 = lax.rem(g4 // JCHUNKS, 2)
                        jj4 = lax.rem(g4, JCHUNKS)
                        pltpu.async_copy(x_hbm.at[c].at[sidx.at[p4].at[jj4]],
                                         rows.at[q], gsems[q])
            return ()

        stage_idx(0, 0)
        wait_idx(0)
        for q in range(4):
            pltpu.async_copy(x_hbm.at[c].at[sidx.at[0].at[q]], rows.at[q], gsems[q])
        lax.fori_loop(0, TOT, chunk_step, ())

    def ident_add(src_hbm):
        def body(k, _):
            pltpu.sync_copy(src_hbm.at[c].at[pl.ds(arow + k * CHUNK, CHUNK)], rows.at[0])
            pltpu.sync_copy(rows.at[0], acc.at[vident.at[k]], add=True)
            return ()
        lax.fori_loop(0, RCHUNKS, body, ())

    barrier = plsc.subcore_barrier

    pltpu.sync_copy(ident.at[pl.ds(s * RCHUNKS, RCHUNKS)], vident)

    bot0 = s * BOT_PER_TILE
    pltpu.sync_copy(botimg.at[c].at[pl.ds(bot0, BOT_PER_TILE)],
                    x3i.at[c].at[pl.ds(USER + bot0, BOT_PER_TILE)])
    pltpu.sync_copy(bottxt.at[c].at[pl.ds(bot0, BOT_PER_TILE)],
                    x3t.at[c].at[pl.ds(USER + bot0, BOT_PER_TILE)])

    zero_acc()
    barrier()
    spmm(asrc, adst, v_adj_w1, ftxt)
    barrier()
    wb_acc(t1)
    wb_top(x3t)
    zero_acc()
    barrier()
    spmm(asrc, adst, v_adj_w0, fimg)
    barrier()
    wb_top(x3i)
    barrier()
    spmm(asrc, adst, v_adj, x3i)
    spmm(isrc, idst, v_iadj, base)
    spmm(asrc, adst, v_adj, x3t)
    spmm(tsrc, tdst, v_tadj, base)
    ident_add(t1)
    barrier()
    wb_acc(m)
    zero_acc()
    barrier()
    spmm(asrc, adst, v_adj, m)
    barrier()
    wb_acc(g1)
    barrier()
    spmm(asrc, adst, v_adj, g1)
    ident_add(m)
    barrier()
    wb_acc(out)


_abst = jax.ShapeDtypeStruct((2, N_PAD, HALF), jnp.float32)
_mega_call = pl.kernel(
    _mega_body,
    out_type=(_abst,) * 6,
    mesh=plsc.VectorSubcoreMesh(core_axis_name="c", subcore_axis_name="s"),
    scratch_types=[
        pltpu.VMEM_SHARED((N_PAD, HALF), jnp.float32),
        pltpu.VMEM((2, JCHUNKS, CHUNK), jnp.int32),
        pltpu.VMEM((2, JCHUNKS, CHUNK), jnp.int32),
        pltpu.VMEM((2, JCHUNKS, CHUNK), jnp.float32),
        pltpu.VMEM((4, CHUNK, HALF), jnp.float32),
        pltpu.VMEM((RCHUNKS, CHUNK), jnp.int32),
    ] + [pltpu.SemaphoreType.DMA] * 8,
    compiler_params=pltpu.CompilerParams(use_tc_tiling_on_sc=False),
)


def _prep_idx(idx):
    pad = E_PAD - E
    fill = jnp.arange(pad, dtype=jnp.int32)
    src = jnp.concatenate([idx[1], fill % N]).reshape(-1, CHUNK)
    dst = jnp.concatenate([idx[0], N + fill % (N_PAD - N)]).reshape(-1, CHUNK)
    return src, dst


def _prep_val(vals):
    return jnp.pad(vals, (0, E_PAD - E)).reshape(-1, CHUNK)


def _to_half(x):
    return x.reshape(x.shape[0], 2, HALF).transpose(1, 0, 2)


def _from_half(x2):
    return x2.transpose(1, 0, 2).reshape(x2.shape[1], LATDIM)


def _l2norm(x):
    n = jnp.linalg.norm(x, axis=1, keepdims=True)
    return x / jnp.maximum(n, 1e-12)


def _final_kernel(pre_ref, m_ref, o_ref):
    mm = m_ref[...]
    sq = jnp.sum(mm * mm, axis=1, keepdims=True)
    o_ref[...] = pre_ref[...] + 0.2 * (mm / jnp.maximum(jnp.sqrt(sq), 1e-12))


def kernel(uEmbeds, iEmbeds, image_embedding, text_embedding, W_img, b_img, W_txt, b_txt,
           modal_weight, adj_vals, image_adj_vals, text_adj_vals,
           adj_idx, image_adj_idx, text_adj_idx):
    weight = jax.nn.softmax(modal_weight, axis=0)
    w0, w1 = weight[0], weight[1]

    asrc, adst = _prep_idx(adj_idx)
    isrc, idst = _prep_idx(image_adj_idx)
    tsrc, tdst = _prep_idx(text_adj_idx)
    v_adj = _prep_val(adj_vals)
    v_adj_w0 = _prep_val(adj_vals * w0)
    v_adj_w1 = _prep_val(adj_vals * w1)
    v_iadj = _prep_val(image_adj_vals * (0.2 * w0))
    v_tadj = _prep_val(text_adj_vals * (0.2 * w1))

    image_feats = image_embedding @ W_img + b_img
    text_feats = text_embedding @ W_txt + b_txt

    u2 = _to_half(uEmbeds)
    i2 = _to_half(iEmbeds)
    base2 = jnp.concatenate([u2, i2], axis=1)
    fimg = jnp.concatenate([u2, _to_half(_l2norm(image_feats))], axis=1)
    ftxt = jnp.concatenate([u2, _to_half(_l2norm(text_feats))], axis=1)
    botimg = jnp.pad(w0 * i2, ((0, 0), (0, BOT_PAD - ITEM), (0, 0)))
    bottxt = jnp.pad(w1 * i2, ((0, 0), (0, BOT_PAD - ITEM), (0, 0)))
    zeros = jnp.zeros((N_PAD, HALF), jnp.float32)
    ident = jnp.arange(N_PAD, dtype=jnp.int32).reshape(-1, CHUNK)

    out2, m2, _, _, _, _ = _mega_call(
        fimg, ftxt, base2, botimg, bottxt, zeros, ident,
        asrc, adst, isrc, idst, tsrc, tdst,
        v_adj, v_adj_w0, v_adj_w1, v_iadj, v_tadj)

    pre = _from_half(out2[:, :N])
    mfull = _from_half(m2[:, :N])

    BM = 1000
    embeds = pl.pallas_call(
        _final_kernel,
        grid=(N // BM,),
        in_specs=[pl.BlockSpec((BM, LATDIM), lambda i: (i, 0))] * 2,
        out_specs=pl.BlockSpec((BM, LATDIM), lambda i: (i, 0)),
        out_shape=jax.ShapeDtypeStruct((N, LATDIM), jnp.float32),
    )(pre, mfull)
    return embeds[:USER], embeds[USER:]
